# baseline (speedup 1.0000x reference)
import functools

import jax
import jax.numpy as jnp
from jax import lax
from jax.experimental import pallas as pl
from jax.experimental.pallas import tpu as pltpu
from jax.experimental.pallas import tpu_sc as plsc

NC = 2
NS = 16
NW = NC * NS
CHUNK = 128
N = 10000
NP = 10240
D = 128
B = 64
ROWS_PER_SUB = NP // NS

_F32 = jnp.float32
_HIGH = lax.Precision.DEFAULT


def _mm_t(a, w):
    return lax.dot_general(a, w, dimension_numbers=(((1,), (1,)), ((), ())),
                           precision=_HIGH, preferred_element_type=_F32)


G = 4


def _sc_segment_sum(m, idx5, zrows, n_groups):
    mesh = plsc.VectorSubcoreMesh(core_axis_name="c", subcore_axis_name="s")

    @functools.partial(
        pl.kernel,
        out_type=jax.ShapeDtypeStruct((NC, NP, D), _F32),
        mesh=mesh,
        scratch_types=[
            pltpu.VMEM((2, G, CHUNK), jnp.int32),
            pltpu.VMEM((2, G, CHUNK), jnp.int32),
            pltpu.VMEM((CHUNK, D), _F32),
            pltpu.VMEM((CHUNK, D), _F32),
            pltpu.VMEM_SHARED((NP, D), _F32),
            pltpu.SemaphoreType.DMA,
            pltpu.SemaphoreType.DMA,
            pltpu.SemaphoreType.DMA,
            pltpu.SemaphoreType.DMA,
        ],
    )
    def k(m_hbm, idx_hbm, z_hbm, out_hbm, ibuf0, ibuf1, rows0, rows1,
          acc_sh, sem_r0, sem_r1, sem_i0, sem_i1):
        cid = lax.axis_index("c")
        sid = lax.axis_index("s")
        wid = sid * NC + cid
        my_idx = idx_hbm.at[wid]

        pltpu.async_copy(my_idx.at[0], ibuf0, sem_i0)
        pltpu.async_copy(my_idx.at[1], ibuf1, sem_i1)
        pltpu.sync_copy(z_hbm, acc_sh.at[pl.ds(sid * ROWS_PER_SUB, ROWS_PER_SUB)])
        pltpu.make_async_copy(my_idx.at[0], ibuf0, sem_i0).wait()
        pltpu.async_copy(m_hbm.at[ibuf0.at[0].at[0]], rows0, sem_r0)
        plsc.subcore_barrier()

        bufs = ((rows0, sem_r0), (rows1, sem_r1))

        def process(ibuf, nxt_ibuf, nxt_sem):
            for c in range(G):
                buf, sem = bufs[c % 2]
                nbuf, nsem = bufs[(c + 1) % 2]
                pltpu.make_async_copy(m_hbm.at[ibuf.at[0].at[0]], buf, sem).wait()
                if c + 1 < G:
                    pltpu.async_copy(m_hbm.at[ibuf.at[0].at[c + 1]], nbuf, nsem)
                else:
                    pltpu.make_async_copy(my_idx.at[0], nxt_ibuf, nxt_sem).wait()
                    pltpu.async_copy(m_hbm.at[nxt_ibuf.at[0].at[0]], nbuf, nsem)
                pltpu.sync_copy(buf, acc_sh.at[ibuf.at[1].at[c]], add=True)

        @pl.loop(0, n_groups, step=2)
        def _(g):
            process(ibuf0, ibuf1, sem_i1)
            pltpu.async_copy(my_idx.at[g + 2], ibuf0, sem_i0)
            process(ibuf1, ibuf0, sem_i0)
            pltpu.async_copy(my_idx.at[g + 3], ibuf1, sem_i1)

        pltpu.make_async_copy(m_hbm.at[ibuf0.at[0].at[0]], rows0, sem_r0).wait()
        pltpu.make_async_copy(my_idx.at[0], ibuf1, sem_i1).wait()
        plsc.subcore_barrier()
        pltpu.sync_copy(
            acc_sh.at[pl.ds(sid * ROWS_PER_SUB, ROWS_PER_SUB)],
            out_hbm.at[cid].at[pl.ds(sid * ROWS_PER_SUB, ROWS_PER_SUB)])

    return k(m, idx5, zrows)


def _tc_pre(h, wr, br, wt):
    def body(h_ref, wr_ref, br_ref, wt_ref, m_ref, r_ref):
        hv = h_ref[...]
        m_ref[...] = _mm_t(hv, wr_ref[...])
        r_ref[...] = _mm_t(hv, wt_ref[...]) + br_ref[...]

    return pl.pallas_call(
        body,
        out_shape=(jax.ShapeDtypeStruct((NP, D), _F32),
                   jax.ShapeDtypeStruct((NP, D), _F32)),
    )(h, wr, br.reshape(1, D), wt)


def _tc_mid(acc, r, wr, br, wt):
    def body(acc_ref, r_ref, wr_ref, br_ref, wt_ref, m_ref, ro_ref):
        h = jnp.maximum(acc_ref[0] + acc_ref[1] + r_ref[...], 0.0)
        m_ref[...] = _mm_t(h, wr_ref[...])
        ro_ref[...] = _mm_t(h, wt_ref[...]) + br_ref[...]

    return pl.pallas_call(
        body,
        out_shape=(jax.ShapeDtypeStruct((NP, D), _F32),
                   jax.ShapeDtypeStruct((NP, D), _F32)),
    )(acc, r, wr, br.reshape(1, D), wt)


def _tc_final(acc, r, batch_row, w1, b1, w2, b2):
    def body(acc_ref, r_ref, b_ref, w1_ref, b1_ref, w2_ref, b2_ref, y_ref):
        h = jnp.maximum(acc_ref[0] + acc_ref[1] + r_ref[...], 0.0)
        seg = b_ref[...]
        mask = (lax.broadcasted_iota(jnp.int32, (B, NP), 0) == seg)
        pooled = lax.dot_general(mask.astype(_F32), h,
                                 dimension_numbers=(((1,), (0,)), ((), ())),
                                 precision=_HIGH, preferred_element_type=_F32)
        t = jnp.maximum(_mm_t(pooled, w1_ref[...]) + b1_ref[...], 0.0)
        y_ref[...] = jnp.sum(t * w2_ref[...], axis=1, keepdims=True) + b2_ref[...]

    return pl.pallas_call(
        body,
        out_shape=jax.ShapeDtypeStruct((B, 1), _F32),
    )(acc, r, batch_row, w1, b1.reshape(1, D), w2, b2.reshape(1, 1))


def kernel(x, edge_index, batch,
           W_rel_0, b_rel_0, W_root_0,
           W_rel_1, b_rel_1, W_root_1,
           W_rel_2, b_rel_2, W_root_2,
           W1, b1, W2, b2):
    e = edge_index.shape[1]
    n_groups = -(-e // (NW * G * CHUNK))
    n_groups += n_groups % 2
    e_pad = NW * n_groups * G * CHUNK

    src = edge_index[0].astype(jnp.int32)
    dst = edge_index[1].astype(jnp.int32)
    src5 = jnp.concatenate(
        [src, jnp.zeros((e_pad - e,), jnp.int32)]
    ).reshape(NW, n_groups, 1, G, CHUNK)
    dst_pad = N + (jnp.arange(e_pad - e, dtype=jnp.int32) % (NP - N))
    dst5 = jnp.concatenate([dst, dst_pad]).reshape(NW, n_groups, 1, G, CHUNK)
    idx5 = jnp.concatenate([src5, dst5], axis=2)
    idx5 = jnp.concatenate(
        [idx5, jnp.zeros((NW, 2, 2, G, CHUNK), jnp.int32)], axis=1)
    batch_row = jnp.concatenate(
        [batch.astype(jnp.int32), jnp.full((NP - N,), B, jnp.int32)]
    ).reshape(1, NP)
    xp = jnp.concatenate([x, jnp.zeros((NP - N, D), _F32)], axis=0)
    zrows = jnp.zeros((ROWS_PER_SUB, D), _F32)

    m, r = _tc_pre(xp, W_rel_0, b_rel_0, W_root_0)
    acc = _sc_segment_sum(m, idx5, zrows, n_groups)
    m, r = _tc_mid(acc, r, W_rel_1, b_rel_1, W_root_1)
    acc = _sc_segment_sum(m, idx5, zrows, n_groups)
    m, r = _tc_mid(acc, r, W_rel_2, b_rel_2, W_root_2)
    acc = _sc_segment_sum(m, idx5, zrows, n_groups)
    return _tc_final(acc, r, batch_row, W1, b1, W2, b2)


# speedup vs baseline: 1.8348x; 1.8348x over previous
import functools

import jax
import jax.numpy as jnp
from jax import lax
from jax.experimental import pallas as pl
from jax.experimental.pallas import tpu as pltpu
from jax.experimental.pallas import tpu_sc as plsc

NC = 2
NS = 16
NW = NC * NS
CHUNK = 128
N = 10000
NP = 10240
D = 128
B = 64
ROWS_PER_SUB = NP // NS

_F32 = jnp.float32
_HIGH = lax.Precision.DEFAULT


def _mm_t(a, w):
    return lax.dot_general(a, w, dimension_numbers=(((1,), (1,)), ((), ())),
                           precision=_HIGH, preferred_element_type=_F32)


def _sc_segment_sum(m, idx4, zrows, n_chunks):
    mesh = plsc.VectorSubcoreMesh(core_axis_name="c", subcore_axis_name="s")

    @functools.partial(
        pl.kernel,
        out_type=jax.ShapeDtypeStruct((NC, NP, D), _F32),
        mesh=mesh,
        scratch_types=[
            pltpu.VMEM((2, n_chunks, CHUNK), jnp.int32),
            pltpu.VMEM((CHUNK, D), _F32),
            pltpu.VMEM_SHARED((NP, D), _F32),
            pltpu.SemaphoreType.DMA,
        ],
    )
    def k(m_hbm, idx_hbm, z_hbm, out_hbm, idx_v, rows_v, acc_sh, sem):
        cid = lax.axis_index("c")
        sid = lax.axis_index("s")
        wid = sid * NC + cid
        pltpu.sync_copy(idx_hbm.at[wid], idx_v)
        pltpu.sync_copy(z_hbm, acc_sh.at[pl.ds(sid * ROWS_PER_SUB, ROWS_PER_SUB)])
        plsc.subcore_barrier()

        @pl.loop(0, n_chunks)
        def _(j):
            pltpu.async_copy(m_hbm.at[idx_v.at[0].at[j]], rows_v, sem).wait()
            pltpu.sync_copy(rows_v, acc_sh.at[idx_v.at[1].at[j]], add=True)

        plsc.subcore_barrier()
        pltpu.sync_copy(
            acc_sh.at[pl.ds(sid * ROWS_PER_SUB, ROWS_PER_SUB)],
            out_hbm.at[cid].at[pl.ds(sid * ROWS_PER_SUB, ROWS_PER_SUB)])

    return k(m, idx4, zrows)


def _tc_pre(h, wr, br, wt):
    def body(h_ref, wr_ref, br_ref, wt_ref, m_ref, r_ref):
        hv = h_ref[...]
        m_ref[...] = _mm_t(hv, wr_ref[...])
        r_ref[...] = _mm_t(hv, wt_ref[...]) + br_ref[...]

    return pl.pallas_call(
        body,
        out_shape=(jax.ShapeDtypeStruct((NP, D), _F32),
                   jax.ShapeDtypeStruct((NP, D), _F32)),
    )(h, wr, br.reshape(1, D), wt)


def _tc_mid(acc, r, wr, br, wt):
    def body(acc_ref, r_ref, wr_ref, br_ref, wt_ref, m_ref, ro_ref):
        h = jnp.maximum(acc_ref[0] + acc_ref[1] + r_ref[...], 0.0)
        m_ref[...] = _mm_t(h, wr_ref[...])
        ro_ref[...] = _mm_t(h, wt_ref[...]) + br_ref[...]

    return pl.pallas_call(
        body,
        out_shape=(jax.ShapeDtypeStruct((NP, D), _F32),
                   jax.ShapeDtypeStruct((NP, D), _F32)),
    )(acc, r, wr, br.reshape(1, D), wt)


def _tc_final(acc, r, batch_row, w1, b1, w2, b2):
    def body(acc_ref, r_ref, b_ref, w1_ref, b1_ref, w2_ref, b2_ref, y_ref):
        h = jnp.maximum(acc_ref[0] + acc_ref[1] + r_ref[...], 0.0)
        seg = b_ref[...]
        mask = (lax.broadcasted_iota(jnp.int32, (B, NP), 0) == seg)
        pooled = lax.dot_general(mask.astype(_F32), h,
                                 dimension_numbers=(((1,), (0,)), ((), ())),
                                 precision=_HIGH, preferred_element_type=_F32)
        t = jnp.maximum(_mm_t(pooled, w1_ref[...]) + b1_ref[...], 0.0)
        y_ref[...] = jnp.sum(t * w2_ref[...], axis=1, keepdims=True) + b2_ref[...]

    return pl.pallas_call(
        body,
        out_shape=jax.ShapeDtypeStruct((B, 1), _F32),
    )(acc, r, batch_row, w1, b1.reshape(1, D), w2, b2.reshape(1, 1))


def kernel(x, edge_index, batch,
           W_rel_0, b_rel_0, W_root_0,
           W_rel_1, b_rel_1, W_root_1,
           W_rel_2, b_rel_2, W_root_2,
           W1, b1, W2, b2):
    e = edge_index.shape[1]
    n_chunks = -(-e // (NW * CHUNK))
    e_pad = NW * n_chunks * CHUNK

    src = edge_index[0].astype(jnp.int32)
    dst = edge_index[1].astype(jnp.int32)
    src4 = jnp.concatenate(
        [src, jnp.zeros((e_pad - e,), jnp.int32)]
    ).reshape(NW, 1, n_chunks, CHUNK)
    dst_pad = N + (jnp.arange(e_pad - e, dtype=jnp.int32) % (NP - N))
    dst4 = jnp.concatenate([dst, dst_pad]).reshape(NW, 1, n_chunks, CHUNK)
    idx4 = jnp.concatenate([src4, dst4], axis=1)
    batch_row = jnp.concatenate(
        [batch.astype(jnp.int32), jnp.full((NP - N,), B, jnp.int32)]
    ).reshape(1, NP)
    xp = jnp.concatenate([x, jnp.zeros((NP - N, D), _F32)], axis=0)
    zrows = jnp.zeros((ROWS_PER_SUB, D), _F32)

    m, r = _tc_pre(xp, W_rel_0, b_rel_0, W_root_0)
    acc = _sc_segment_sum(m, idx4, zrows, n_chunks)
    m, r = _tc_mid(acc, r, W_rel_1, b_rel_1, W_root_1)
    acc = _sc_segment_sum(m, idx4, zrows, n_chunks)
    m, r = _tc_mid(acc, r, W_rel_2, b_rel_2, W_root_2)
    acc = _sc_segment_sum(m, idx4, zrows, n_chunks)
    return _tc_final(acc, r, batch_row, W1, b1, W2, b2)


# speedup vs baseline: 2.6842x; 1.4630x over previous
import functools

import jax
import jax.numpy as jnp
from jax import lax
from jax.experimental import pallas as pl
from jax.experimental.pallas import tpu as pltpu
from jax.experimental.pallas import tpu_sc as plsc

NC = 2
NS = 16
NW = NC * NS
CHUNK = 128
N = 10000
NP = 10240
D = 128
B = 64
ROWS_PER_SUB = NP // NS

_F32 = jnp.float32
_HIGH = lax.Precision.DEFAULT


def _mm_t(a, w):
    return lax.dot_general(a, w, dimension_numbers=(((1,), (1,)), ((), ())),
                           precision=_HIGH, preferred_element_type=_F32)


C_EVEN = 101
C_ODD = 56
MAXC = max(C_EVEN, C_ODD)


def _sc_segment_sum(m, idx4, zrows):
    mesh = plsc.VectorSubcoreMesh(core_axis_name="c", subcore_axis_name="s")

    @functools.partial(
        pl.kernel,
        out_type=jax.ShapeDtypeStruct((NC, NP, D), _F32),
        mesh=mesh,
        scratch_types=[
            pltpu.VMEM((2, MAXC, CHUNK), jnp.int32),
            pltpu.VMEM((CHUNK, D), _F32),
            pltpu.VMEM_SHARED((NP, D), _F32),
            pltpu.SemaphoreType.DMA,
        ],
    )
    def k(m_hbm, idx_hbm, z_hbm, out_hbm, idx_v, rows_v, acc_sh, sem):
        cid = lax.axis_index("c")
        sid = lax.axis_index("s")
        wid = sid * NC + cid
        n_mine = jnp.where(cid == 0, C_EVEN, C_ODD)
        pltpu.sync_copy(idx_hbm.at[wid], idx_v)
        pltpu.sync_copy(z_hbm, acc_sh.at[pl.ds(sid * ROWS_PER_SUB, ROWS_PER_SUB)])
        plsc.subcore_barrier()

        @pl.loop(0, n_mine)
        def _(j):
            pltpu.async_copy(m_hbm.at[idx_v.at[0].at[j]], rows_v, sem).wait()
            pltpu.sync_copy(rows_v, acc_sh.at[idx_v.at[1].at[j]], add=True)

        plsc.subcore_barrier()
        pltpu.sync_copy(
            acc_sh.at[pl.ds(sid * ROWS_PER_SUB, ROWS_PER_SUB)],
            out_hbm.at[cid].at[pl.ds(sid * ROWS_PER_SUB, ROWS_PER_SUB)])

    return k(m, idx4, zrows)


def _tc_pre(h, wr, br, wt):
    def body(h_ref, wr_ref, br_ref, wt_ref, m_ref, r_ref):
        hv = h_ref[...]
        m_ref[...] = _mm_t(hv, wr_ref[...])
        r_ref[...] = _mm_t(hv, wt_ref[...]) + br_ref[...]

    return pl.pallas_call(
        body,
        out_shape=(jax.ShapeDtypeStruct((NP, D), _F32),
                   jax.ShapeDtypeStruct((NP, D), _F32)),
    )(h, wr, br.reshape(1, D), wt)


def _tc_mid(acc, r, wr, br, wt):
    def body(acc_ref, r_ref, wr_ref, br_ref, wt_ref, m_ref, ro_ref):
        h = jnp.maximum(acc_ref[0] + acc_ref[1] + r_ref[...], 0.0)
        m_ref[...] = _mm_t(h, wr_ref[...])
        ro_ref[...] = _mm_t(h, wt_ref[...]) + br_ref[...]

    return pl.pallas_call(
        body,
        out_shape=(jax.ShapeDtypeStruct((NP, D), _F32),
                   jax.ShapeDtypeStruct((NP, D), _F32)),
    )(acc, r, wr, br.reshape(1, D), wt)


def _tc_final(acc, r, batch_row, w1, b1, w2, b2):
    def body(acc_ref, r_ref, b_ref, w1_ref, b1_ref, w2_ref, b2_ref, y_ref):
        h = jnp.maximum(acc_ref[0] + acc_ref[1] + r_ref[...], 0.0)
        seg = b_ref[...]
        mask = (lax.broadcasted_iota(jnp.int32, (B, NP), 0) == seg)
        pooled = lax.dot_general(mask.astype(_F32), h,
                                 dimension_numbers=(((1,), (0,)), ((), ())),
                                 precision=lax.Precision.HIGHEST,
                                 preferred_element_type=_F32)
        t = jnp.maximum(_mm_t(pooled, w1_ref[...]) + b1_ref[...], 0.0)
        y_ref[...] = jnp.sum(t * w2_ref[...], axis=1, keepdims=True) + b2_ref[...]

    return pl.pallas_call(
        body,
        out_shape=jax.ShapeDtypeStruct((B, 1), _F32),
    )(acc, r, batch_row, w1, b1.reshape(1, D), w2, b2.reshape(1, 1))


def kernel(x, edge_index, batch,
           W_rel_0, b_rel_0, W_root_0,
           W_rel_1, b_rel_1, W_root_1,
           W_rel_2, b_rel_2, W_root_2,
           W1, b1, W2, b2):
    e = edge_index.shape[1]
    e_pad = NS * (C_EVEN + C_ODD) * CHUNK
    assert e_pad >= e

    src = edge_index[0].astype(jnp.int32)
    dst = edge_index[1].astype(jnp.int32)

    def to_workers(flat):
        chunks = flat.reshape(-1, CHUNK)
        a = chunks[:NS * C_EVEN].reshape(NS, C_EVEN, CHUNK)
        b = chunks[NS * C_EVEN:].reshape(NS, C_ODD, CHUNK)
        a = jnp.concatenate(
            [a, jnp.zeros((NS, MAXC - C_EVEN, CHUNK), jnp.int32)], axis=1)
        b = jnp.concatenate(
            [b, jnp.zeros((NS, MAXC - C_ODD, CHUNK), jnp.int32)], axis=1)
        return jnp.stack([a, b], axis=1).reshape(NW, MAXC, CHUNK)

    dst_pad = N + (jnp.arange(e_pad - e, dtype=jnp.int32) % (NP - N))
    src4 = to_workers(jnp.concatenate([src, jnp.zeros((e_pad - e,), jnp.int32)]))
    dst4 = to_workers(jnp.concatenate([dst, dst_pad]))
    idx4 = jnp.stack([src4, dst4], axis=1)
    batch_row = jnp.concatenate(
        [batch.astype(jnp.int32), jnp.full((NP - N,), B, jnp.int32)]
    ).reshape(1, NP)
    xp = jnp.concatenate([x, jnp.zeros((NP - N, D), _F32)], axis=0)
    zrows = jnp.zeros((ROWS_PER_SUB, D), _F32)

    m, r = _tc_pre(xp, W_rel_0, b_rel_0, W_root_0)
    acc = _sc_segment_sum(m, idx4, zrows)
    m, r = _tc_mid(acc, r, W_rel_1, b_rel_1, W_root_1)
    acc = _sc_segment_sum(m, idx4, zrows)
    m, r = _tc_mid(acc, r, W_rel_2, b_rel_2, W_root_2)
    acc = _sc_segment_sum(m, idx4, zrows)
    return _tc_final(acc, r, batch_row, W1, b1, W2, b2)


# speedup vs baseline: 2.6977x; 1.0050x over previous
import functools

import jax
import jax.numpy as jnp
from jax import lax
from jax.experimental import pallas as pl
from jax.experimental.pallas import tpu as pltpu
from jax.experimental.pallas import tpu_sc as plsc

NC = 2
NS = 16
NW = NC * NS
CHUNK = 128
N = 10000
NP = 10240
D = 128
B = 64
ROWS_PER_SUB = NP // NS

_F32 = jnp.float32
_HIGH = lax.Precision.DEFAULT


def _mm_t(a, w):
    return lax.dot_general(a, w, dimension_numbers=(((1,), (1,)), ((), ())),
                           precision=_HIGH, preferred_element_type=_F32)


C_EVEN = 101
C_ODD = 56
MAXC = max(C_EVEN, C_ODD)


def _sc_segment_sum(m, idx4, zrows):
    mesh = plsc.VectorSubcoreMesh(core_axis_name="c", subcore_axis_name="s")

    @functools.partial(
        pl.kernel,
        out_type=jax.ShapeDtypeStruct((NC, NP, D), _F32),
        mesh=mesh,
        scratch_types=[
            pltpu.VMEM((2, MAXC, CHUNK), jnp.int32),
            pltpu.VMEM((CHUNK, D), _F32),
            pltpu.VMEM_SHARED((NP, D), _F32),
            pltpu.SemaphoreType.DMA,
        ],
    )
    def k(m_hbm, idx_hbm, z_hbm, out_hbm, idx_v, rows_v, acc_sh, sem):
        cid = lax.axis_index("c")
        sid = lax.axis_index("s")
        wid = sid * NC + cid
        n_mine = jnp.where(cid == 0, C_EVEN, C_ODD)
        pltpu.sync_copy(idx_hbm.at[wid], idx_v)
        pltpu.sync_copy(z_hbm, acc_sh.at[pl.ds(sid * ROWS_PER_SUB, ROWS_PER_SUB)])
        plsc.subcore_barrier()

        @pl.loop(0, n_mine)
        def _(j):
            pltpu.async_copy(m_hbm.at[idx_v.at[0].at[j]], rows_v, sem).wait()
            pltpu.sync_copy(rows_v, acc_sh.at[idx_v.at[1].at[j]], add=True)

        plsc.subcore_barrier()
        pltpu.sync_copy(
            acc_sh.at[pl.ds(sid * ROWS_PER_SUB, ROWS_PER_SUB)],
            out_hbm.at[cid].at[pl.ds(sid * ROWS_PER_SUB, ROWS_PER_SUB)])

    return k(m, idx4, zrows)


def _tc_layer(acc, h, wr, br, wt):
    def body(acc_ref, h_ref, wr_ref, br_ref, wt_ref, o_ref):
        agg = acc_ref[0] + acc_ref[1]
        o_ref[...] = jnp.maximum(
            _mm_t(agg, wr_ref[...]) + br_ref[...] + _mm_t(h_ref[...], wt_ref[...]),
            0.0)

    return pl.pallas_call(
        body,
        out_shape=jax.ShapeDtypeStruct((NP, D), _F32),
    )(acc, h, wr, br.reshape(1, D), wt)


def _tc_final(acc, h_in, wr, br, wt, batch_row, w1, b1, w2, b2):
    def body(acc_ref, h_ref, wr_ref, br_ref, wt_ref, b_ref,
             w1_ref, b1_ref, w2_ref, b2_ref, y_ref):
        agg = acc_ref[0] + acc_ref[1]
        h = jnp.maximum(
            _mm_t(agg, wr_ref[...]) + br_ref[...] + _mm_t(h_ref[...], wt_ref[...]),
            0.0)
        seg = b_ref[...]
        mask = (lax.broadcasted_iota(jnp.int32, (B, NP), 0) == seg)
        pooled = lax.dot_general(mask.astype(_F32), h,
                                 dimension_numbers=(((1,), (0,)), ((), ())),
                                 precision=lax.Precision.HIGHEST,
                                 preferred_element_type=_F32)
        t = jnp.maximum(_mm_t(pooled, w1_ref[...]) + b1_ref[...], 0.0)
        y_ref[...] = jnp.sum(t * w2_ref[...], axis=1, keepdims=True) + b2_ref[...]

    return pl.pallas_call(
        body,
        out_shape=jax.ShapeDtypeStruct((B, 1), _F32),
    )(acc, h_in, wr, br.reshape(1, D), wt, batch_row,
      w1, b1.reshape(1, D), w2, b2.reshape(1, 1))


def kernel(x, edge_index, batch,
           W_rel_0, b_rel_0, W_root_0,
           W_rel_1, b_rel_1, W_root_1,
           W_rel_2, b_rel_2, W_root_2,
           W1, b1, W2, b2):
    e = edge_index.shape[1]
    e_pad = NS * (C_EVEN + C_ODD) * CHUNK
    assert e_pad >= e

    src = edge_index[0].astype(jnp.int32)
    dst = edge_index[1].astype(jnp.int32)

    def to_workers(flat):
        chunks = flat.reshape(-1, CHUNK)
        a = chunks[:NS * C_EVEN].reshape(NS, C_EVEN, CHUNK)
        b = chunks[NS * C_EVEN:].reshape(NS, C_ODD, CHUNK)
        a = jnp.concatenate(
            [a, jnp.zeros((NS, MAXC - C_EVEN, CHUNK), jnp.int32)], axis=1)
        b = jnp.concatenate(
            [b, jnp.zeros((NS, MAXC - C_ODD, CHUNK), jnp.int32)], axis=1)
        return jnp.stack([a, b], axis=1).reshape(NW, MAXC, CHUNK)

    dst_pad = N + (jnp.arange(e_pad - e, dtype=jnp.int32) % (NP - N))
    src4 = to_workers(jnp.concatenate([src, jnp.zeros((e_pad - e,), jnp.int32)]))
    dst4 = to_workers(jnp.concatenate([dst, dst_pad]))
    idx4 = jnp.stack([src4, dst4], axis=1)
    batch_row = jnp.concatenate(
        [batch.astype(jnp.int32), jnp.full((NP - N,), B, jnp.int32)]
    ).reshape(1, NP)
    xp = jnp.concatenate([x, jnp.zeros((NP - N, D), _F32)], axis=0)
    zrows = jnp.zeros((ROWS_PER_SUB, D), _F32)

    acc = _sc_segment_sum(xp, idx4, zrows)
    h = _tc_layer(acc, xp, W_rel_0, b_rel_0, W_root_0)
    acc = _sc_segment_sum(h, idx4, zrows)
    h = _tc_layer(acc, h, W_rel_1, b_rel_1, W_root_1)
    acc = _sc_segment_sum(h, idx4, zrows)
    return _tc_final(acc, h, W_rel_2, b_rel_2, W_root_2,
                     batch_row, W1, b1, W2, b2)


# speedup vs baseline: 2.8145x; 1.0433x over previous
import functools

import jax
import jax.numpy as jnp
from jax import lax
from jax.experimental import pallas as pl
from jax.experimental.pallas import tpu as pltpu
from jax.experimental.pallas import tpu_sc as plsc

NC = 2
NS = 16
NW = NC * NS
CHUNK = 128
N = 10000
NP = 10240
D = 128
B = 64
ROWS_PER_SUB = NP // NS

_F32 = jnp.float32
_HIGH = lax.Precision.DEFAULT


def _mm_t(a, w):
    return lax.dot_general(a, w, dimension_numbers=(((1,), (1,)), ((), ())),
                           precision=_HIGH, preferred_element_type=_F32)


C_EVEN = 95
C_ODD = 62
MAXC = max(C_EVEN, C_ODD)


def _sc_segment_sum(m, idx4, zrows):
    mesh = plsc.VectorSubcoreMesh(core_axis_name="c", subcore_axis_name="s")

    @functools.partial(
        pl.kernel,
        out_type=jax.ShapeDtypeStruct((NC, NP, D), _F32),
        mesh=mesh,
        scratch_types=[
            pltpu.VMEM((2, MAXC, CHUNK), jnp.int32),
            pltpu.VMEM((CHUNK, D), _F32),
            pltpu.VMEM_SHARED((NP, D), _F32),
            pltpu.SemaphoreType.DMA,
        ],
    )
    def k(m_hbm, idx_hbm, z_hbm, out_hbm, idx_v, rows_v, acc_sh, sem):
        cid = lax.axis_index("c")
        sid = lax.axis_index("s")
        wid = sid * NC + cid
        n_mine = jnp.where(cid == 0, C_EVEN, C_ODD)
        pltpu.sync_copy(idx_hbm.at[wid], idx_v)
        pltpu.sync_copy(z_hbm, acc_sh.at[pl.ds(sid * ROWS_PER_SUB, ROWS_PER_SUB)])
        plsc.subcore_barrier()

        @pl.loop(0, n_mine)
        def _(j):
            pltpu.async_copy(m_hbm.at[idx_v.at[0].at[j]], rows_v, sem).wait()
            pltpu.sync_copy(rows_v, acc_sh.at[idx_v.at[1].at[j]], add=True)

        plsc.subcore_barrier()
        pltpu.sync_copy(
            acc_sh.at[pl.ds(sid * ROWS_PER_SUB, ROWS_PER_SUB)],
            out_hbm.at[cid].at[pl.ds(sid * ROWS_PER_SUB, ROWS_PER_SUB)])

    return k(m, idx4, zrows)


def _tc_layer(acc, h, wr, br, wt):
    def body(acc_ref, h_ref, wr_ref, br_ref, wt_ref, o_ref):
        agg = acc_ref[0] + acc_ref[1]
        o_ref[...] = jnp.maximum(
            _mm_t(agg, wr_ref[...]) + br_ref[...] + _mm_t(h_ref[...], wt_ref[...]),
            0.0)

    return pl.pallas_call(
        body,
        out_shape=jax.ShapeDtypeStruct((NP, D), _F32),
    )(acc, h, wr, br.reshape(1, D), wt)


def _tc_final(acc, h_in, wr, br, wt, batch_row, w1, b1, w2, b2):
    def body(acc_ref, h_ref, wr_ref, br_ref, wt_ref, b_ref,
             w1_ref, b1_ref, w2_ref, b2_ref, y_ref):
        agg = acc_ref[0] + acc_ref[1]
        h = jnp.maximum(
            _mm_t(agg, wr_ref[...]) + br_ref[...] + _mm_t(h_ref[...], wt_ref[...]),
            0.0)
        seg = b_ref[...]
        mask = (lax.broadcasted_iota(jnp.int32, (B, NP), 0) == seg)
        pooled = lax.dot_general(mask.astype(_F32), h,
                                 dimension_numbers=(((1,), (0,)), ((), ())),
                                 precision=lax.Precision.HIGHEST,
                                 preferred_element_type=_F32)
        t = jnp.maximum(_mm_t(pooled, w1_ref[...]) + b1_ref[...], 0.0)
        y_ref[...] = jnp.sum(t * w2_ref[...], axis=1, keepdims=True) + b2_ref[...]

    return pl.pallas_call(
        body,
        out_shape=jax.ShapeDtypeStruct((B, 1), _F32),
    )(acc, h_in, wr, br.reshape(1, D), wt, batch_row,
      w1, b1.reshape(1, D), w2, b2.reshape(1, 1))


def kernel(x, edge_index, batch,
           W_rel_0, b_rel_0, W_root_0,
           W_rel_1, b_rel_1, W_root_1,
           W_rel_2, b_rel_2, W_root_2,
           W1, b1, W2, b2):
    e = edge_index.shape[1]
    e_pad = NS * (C_EVEN + C_ODD) * CHUNK
    assert e_pad >= e

    src = edge_index[0].astype(jnp.int32)
    dst = edge_index[1].astype(jnp.int32)

    def to_workers(flat):
        chunks = flat.reshape(-1, CHUNK)
        a = chunks[:NS * C_EVEN].reshape(NS, C_EVEN, CHUNK)
        b = chunks[NS * C_EVEN:].reshape(NS, C_ODD, CHUNK)
        a = jnp.concatenate(
            [a, jnp.zeros((NS, MAXC - C_EVEN, CHUNK), jnp.int32)], axis=1)
        b = jnp.concatenate(
            [b, jnp.zeros((NS, MAXC - C_ODD, CHUNK), jnp.int32)], axis=1)
        return jnp.stack([a, b], axis=1).reshape(NW, MAXC, CHUNK)

    dst_pad = N + (jnp.arange(e_pad - e, dtype=jnp.int32) % (NP - N))
    src4 = to_workers(jnp.concatenate([src, jnp.zeros((e_pad - e,), jnp.int32)]))
    dst4 = to_workers(jnp.concatenate([dst, dst_pad]))
    idx4 = jnp.stack([src4, dst4], axis=1)
    batch_row = jnp.concatenate(
        [batch.astype(jnp.int32), jnp.full((NP - N,), B, jnp.int32)]
    ).reshape(1, NP)
    xp = jnp.concatenate([x, jnp.zeros((NP - N, D), _F32)], axis=0)
    zrows = jnp.zeros((ROWS_PER_SUB, D), _F32)

    acc = _sc_segment_sum(xp, idx4, zrows)
    h = _tc_layer(acc, xp, W_rel_0, b_rel_0, W_root_0)
    acc = _sc_segment_sum(h, idx4, zrows)
    h = _tc_layer(acc, h, W_rel_1, b_rel_1, W_root_1)
    acc = _sc_segment_sum(h, idx4, zrows)
    return _tc_final(acc, h, W_rel_2, b_rel_2, W_root_2,
                     batch_row, W1, b1, W2, b2)
